# baseline (speedup 1.0000x reference)
import jax
import jax.numpy as jnp
from jax.experimental import pallas as pl
from jax.experimental.pallas import tpu as pltpu

B = 32
ZS = 128
CC = 16
N_OUT = 65536
N_CHUNKS = 8
CHUNK = N_OUT // N_CHUNKS

_R_Z = 0
_R_IDX = 32
_R_ES = 64
_R_ET = 128
_R_EC = 256
_R_FC0 = 512
_R_FC1 = 528
_R_FC2 = 560
_R_FC3 = 608
_R_FC4 = 656
_R_UP0 = 704
_R_UP1 = 848
_R_UP2 = 928
_R_UP3 = 976
_R_UP4 = 1008
_R_BIAS = 1032
N_ROWS = 1042


def _leaky(x):
    return jnp.where(x > 0, x, 0.2 * x)


def _fgl_kernel(p_ref, out_ref, y_ref):
    f32 = jnp.float32

    @pl.when(pl.program_id(0) == 0)
    def _compute_chain():
        def onehot(col, n):
            iota = jax.lax.broadcasted_iota(jnp.int32, (B, n), 1)
            return (iota == col.astype(jnp.int32)).astype(f32)

        se = onehot(p_ref[_R_IDX:_R_IDX + B, 0:1], 64) @ p_ref[_R_ES:_R_ES + 64, 0:CC]
        te = onehot(p_ref[_R_IDX:_R_IDX + B, 1:2], 128) @ p_ref[_R_ET:_R_ET + 128, 0:CC]
        ce = onehot(p_ref[_R_IDX:_R_IDX + B, 2:3], 256) @ p_ref[_R_EC:_R_EC + 256, 0:CC]
        cat3 = jnp.concatenate([se, te, ce], axis=1)

        c0 = se @ p_ref[_R_FC0:_R_FC0 + 16, 0:CC] + p_ref[_R_BIAS:_R_BIAS + 1, 0:CC]
        c1 = (jnp.concatenate([se, te], axis=1) @ p_ref[_R_FC1:_R_FC1 + 32, 0:CC]
              + p_ref[_R_BIAS + 1:_R_BIAS + 2, 0:CC])
        c2 = cat3 @ p_ref[_R_FC2:_R_FC2 + 48, 0:CC] + p_ref[_R_BIAS + 2:_R_BIAS + 3, 0:CC]
        c3 = cat3 @ p_ref[_R_FC3:_R_FC3 + 48, 0:CC] + p_ref[_R_BIAS + 3:_R_BIAS + 4, 0:CC]
        c4 = cat3 @ p_ref[_R_FC4:_R_FC4 + 48, 0:CC] + p_ref[_R_BIAS + 4:_R_BIAS + 5, 0:CC]

        x = p_ref[_R_Z:_R_Z + B, :]
        x = _leaky(jnp.concatenate([x, c0], axis=1) @ p_ref[_R_UP0:_R_UP0 + 144, 0:64]
                   + p_ref[_R_BIAS + 5:_R_BIAS + 6, 0:64])
        x = _leaky(jnp.concatenate([x, c1], axis=1) @ p_ref[_R_UP1:_R_UP1 + 80, 0:32]
                   + p_ref[_R_BIAS + 6:_R_BIAS + 7, 0:32])
        x = _leaky(jnp.concatenate([x, c2], axis=1) @ p_ref[_R_UP2:_R_UP2 + 48, 0:CC]
                   + p_ref[_R_BIAS + 7:_R_BIAS + 8, 0:CC])
        x = _leaky(jnp.concatenate([x, c3], axis=1) @ p_ref[_R_UP3:_R_UP3 + 32, 0:8]
                   + p_ref[_R_BIAS + 8:_R_BIAS + 9, 0:8])
        y = (jnp.concatenate([x, c4], axis=1) @ p_ref[_R_UP4:_R_UP4 + 24, 0:1]
             + p_ref[_R_BIAS + 9:_R_BIAS + 10, 0:1])
        y_ref[:, :] = y

    yv = y_ref[:, :]
    out_ref[:, :, :] = jnp.broadcast_to(yv[:, :, None], (B, CHUNK // 128, 128))


def _pad_rows(a):
    return jnp.pad(a, ((0, 0), (0, 128 - a.shape[1])))


def kernel(z, studies, tasks, contrasts, Es, Et, Ec,
           fc0_W, fc0_b, fc1_W, fc1_b, fc2_W, fc2_b, fc3_W, fc3_b,
           fc4_W, fc4_b, up0_W, up0_b, up1_W, up1_b, up2_W, up2_b,
           up3_W, up3_b, up4_W, up4_b, idx0, idx1, idx2, idx3, idx4):
    f32 = jnp.float32
    idx_cols = jnp.stack([studies, tasks, contrasts], axis=1).astype(f32)
    biases = [fc0_b, fc1_b, fc2_b, fc3_b, fc4_b, up0_b, up1_b, up2_b, up3_b, up4_b]
    packed = jnp.concatenate(
        [z, _pad_rows(idx_cols),
         _pad_rows(Es), _pad_rows(Et), _pad_rows(Ec),
         _pad_rows(fc0_W), _pad_rows(fc1_W), _pad_rows(fc2_W),
         _pad_rows(fc3_W), _pad_rows(fc4_W),
         _pad_rows(up0_W), _pad_rows(up1_W), _pad_rows(up2_W),
         _pad_rows(up3_W), _pad_rows(up4_W)]
        + [_pad_rows(b[None, :]) for b in biases],
        axis=0)
    out = pl.pallas_call(
        _fgl_kernel,
        grid=(N_CHUNKS,),
        in_specs=[pl.BlockSpec((N_ROWS, 128), lambda i: (0, 0))],
        out_specs=pl.BlockSpec((B, CHUNK // 128, 128), lambda i: (0, i, 0)),
        out_shape=jax.ShapeDtypeStruct((B, N_OUT // 128, 128), jnp.float32),
        scratch_shapes=[pltpu.VMEM((B, 1), jnp.float32)],
    )(packed)
    return out.reshape(B, N_OUT, 1)


# speedup vs baseline: 1.3020x; 1.3020x over previous
import jax
import jax.numpy as jnp
from jax.experimental import pallas as pl
from jax.experimental.pallas import tpu as pltpu

B = 32
ZS = 128
CC = 16
N_OUT = 65536
N_CHUNKS = 8
CHUNK = N_OUT // N_CHUNKS

_B_FC = [0, 16, 32, 48, 64]
_B_UP = [80, 144, 176, 192, 200]
_UP_OUT = [64, 32, 16, 8, 1]


def _leaky(x):
    return jnp.where(x > 0, x, 0.2 * x)


def _fgl_kernel(idx_hbm, bias_hbm, z_hbm, Es_hbm, Et_hbm, Ec_hbm,
                fc0_hbm, fc1_hbm, fc2_hbm, fc3_hbm, fc4_hbm,
                up0_hbm, up1_hbm, up2_hbm, up3_hbm, up4_hbm,
                out_ref,
                idx_v, bias_v, z_v, Es_v, Et_v, Ec_v,
                fc0_v, fc1_v, fc2_v, fc3_v, fc4_v,
                up0_v, up1_v, up2_v, up3_v, up4_v,
                y_ref, sem):
    f32 = jnp.float32

    @pl.when(pl.program_id(0) == 0)
    def _compute_chain():
        pairs = [(idx_hbm, idx_v), (bias_hbm, bias_v), (z_hbm, z_v),
                 (Es_hbm, Es_v), (Et_hbm, Et_v), (Ec_hbm, Ec_v),
                 (fc0_hbm, fc0_v), (fc1_hbm, fc1_v), (fc2_hbm, fc2_v),
                 (fc3_hbm, fc3_v), (fc4_hbm, fc4_v),
                 (up0_hbm, up0_v), (up1_hbm, up1_v), (up2_hbm, up2_v),
                 (up3_hbm, up3_v), (up4_hbm, up4_v)]
        copies = [pltpu.make_async_copy(s, d, sem.at[i])
                  for i, (s, d) in enumerate(pairs)]
        for c in copies:
            c.start()
        for c in copies:
            c.wait()

        def onehot(col, n):
            iota = jax.lax.broadcasted_iota(jnp.int32, (B, n), 1)
            return (iota == col).astype(f32)

        bias = bias_v[:]

        se = onehot(idx_v[:, 0:1], 64) @ Es_v[:, :]
        te = onehot(idx_v[:, 1:2], 128) @ Et_v[:, :]
        ce = onehot(idx_v[:, 2:3], 256) @ Ec_v[:, :]
        cat3 = jnp.concatenate([se, te, ce], axis=1)

        def fcb(i):
            return jnp.broadcast_to(bias[_B_FC[i]:_B_FC[i] + CC][None, :], (B, CC))

        def upb(i):
            w = _UP_OUT[i]
            return jnp.broadcast_to(bias[_B_UP[i]:_B_UP[i] + w][None, :], (B, w))

        c0 = se @ fc0_v[:, :] + fcb(0)
        c1 = jnp.concatenate([se, te], axis=1) @ fc1_v[:, :] + fcb(1)
        c2 = cat3 @ fc2_v[:, :] + fcb(2)
        c3 = cat3 @ fc3_v[:, :] + fcb(3)
        c4 = cat3 @ fc4_v[:, :] + fcb(4)

        x = z_v[:, :]
        x = _leaky(jnp.concatenate([x, c0], axis=1) @ up0_v[:, :] + upb(0))
        x = _leaky(jnp.concatenate([x, c1], axis=1) @ up1_v[:, :] + upb(1))
        x = _leaky(jnp.concatenate([x, c2], axis=1) @ up2_v[:, :] + upb(2))
        x = _leaky(jnp.concatenate([x, c3], axis=1) @ up3_v[:, :] + upb(3))
        y = jnp.concatenate([x, c4], axis=1) @ up4_v[:, :] + upb(4)
        y_ref[:, :] = y

    yv = y_ref[:, :]
    out_ref[:, :, :] = jnp.broadcast_to(yv[:, :, None], (B, CHUNK // 128, 128))


def kernel(z, studies, tasks, contrasts, Es, Et, Ec,
           fc0_W, fc0_b, fc1_W, fc1_b, fc2_W, fc2_b, fc3_W, fc3_b,
           fc4_W, fc4_b, up0_W, up0_b, up1_W, up1_b, up2_W, up2_b,
           up3_W, up3_b, up4_W, up4_b, idx0, idx1, idx2, idx3, idx4):
    idx2d = jnp.stack([studies, tasks, contrasts], axis=1)
    bias_all = jnp.concatenate([fc0_b, fc1_b, fc2_b, fc3_b, fc4_b,
                                up0_b, up1_b, up2_b, up3_b, up4_b])
    args = (idx2d, bias_all, z, Es, Et, Ec,
            fc0_W, fc1_W, fc2_W, fc3_W, fc4_W,
            up0_W, up1_W, up2_W, up3_W, up4_W)
    hbm = pl.BlockSpec(memory_space=pltpu.MemorySpace.HBM)
    out = pl.pallas_call(
        _fgl_kernel,
        grid=(N_CHUNKS,),
        in_specs=[hbm] * len(args),
        out_specs=pl.BlockSpec((B, CHUNK // 128, 128), lambda i: (0, i, 0)),
        out_shape=jax.ShapeDtypeStruct((B, N_OUT // 128, 128), jnp.float32),
        scratch_shapes=[
            pltpu.VMEM((B, 3), jnp.int32),
            pltpu.VMEM((201,), jnp.float32),
            pltpu.VMEM((B, ZS), jnp.float32),
            pltpu.VMEM((64, CC), jnp.float32),
            pltpu.VMEM((128, CC), jnp.float32),
            pltpu.VMEM((256, CC), jnp.float32),
            pltpu.VMEM((CC, CC), jnp.float32),
            pltpu.VMEM((2 * CC, CC), jnp.float32),
            pltpu.VMEM((3 * CC, CC), jnp.float32),
            pltpu.VMEM((3 * CC, CC), jnp.float32),
            pltpu.VMEM((3 * CC, CC), jnp.float32),
            pltpu.VMEM((ZS + CC, 64), jnp.float32),
            pltpu.VMEM((64 + CC, 32), jnp.float32),
            pltpu.VMEM((32 + CC, 16), jnp.float32),
            pltpu.VMEM((16 + CC, 8), jnp.float32),
            pltpu.VMEM((8 + CC, 1), jnp.float32),
            pltpu.VMEM((B, 1), jnp.float32),
            pltpu.SemaphoreType.DMA((16,)),
        ],
    )(*args)
    return out.reshape(B, N_OUT, 1)


# speedup vs baseline: 3.0851x; 2.3694x over previous
import jax
import jax.numpy as jnp
from jax.experimental import pallas as pl
from jax.experimental.pallas import tpu as pltpu

B = 32
ZS = 128
CC = 16
N_OUT = 65536
N_CHUNKS = 8
CHUNK = N_OUT // N_CHUNKS

_B_FC = [0, 16, 32, 48, 64]
_B_UP = [80, 144, 176, 192, 200]
_UP_OUT = [64, 32, 16, 8, 1]


def _leaky(x):
    return jnp.where(x > 0, x, 0.2 * x)


def _dot_t(a, b_t):
    return jax.lax.dot_general(a, b_t, (((1,), (1,)), ((), ())),
                               preferred_element_type=jnp.float32)


def _fgl_kernel(idx_hbm, bias_hbm, z_hbm, Es_hbm, Et_hbm, Ec_hbm,
                fc0_hbm, fc1_hbm, fc2_hbm, fc3_hbm, fc4_hbm,
                up0_hbm, up1_hbm, up2_hbm, up3_hbm, up4_hbm,
                out_ref,
                idx_v, bias_v, z_v, Es_v, Et_v, Ec_v,
                fc0_v, fc1_v, fc2_v, fc3_v, fc4_v,
                up0_v, up1_v, up2_v, up3_v, up4_v,
                y_ref, sem):
    f32 = jnp.float32

    @pl.when(pl.program_id(0) == 0)
    def _compute_chain():
        pairs = [(idx_hbm, idx_v), (bias_hbm, bias_v), (z_hbm, z_v),
                 (Es_hbm, Es_v), (Et_hbm, Et_v), (Ec_hbm, Ec_v),
                 (fc0_hbm, fc0_v), (fc1_hbm, fc1_v), (fc2_hbm, fc2_v),
                 (fc3_hbm, fc3_v), (fc4_hbm, fc4_v),
                 (up0_hbm, up0_v), (up1_hbm, up1_v), (up2_hbm, up2_v),
                 (up3_hbm, up3_v), (up4_hbm, up4_v)]
        copies = [pltpu.make_async_copy(s, d, sem.at[i])
                  for i, (s, d) in enumerate(pairs)]
        for c in copies:
            c.start()
        for c in copies:
            c.wait()

        def onehot(col, n):
            iota = jax.lax.broadcasted_iota(jnp.int32, (B, n), 1)
            return (iota == col).astype(f32)

        bias = bias_v[:]

        se = _dot_t(onehot(idx_v[:, 0:1], 64), Es_v[:, :])
        te = _dot_t(onehot(idx_v[:, 1:2], 128), Et_v[:, :])
        ce = _dot_t(onehot(idx_v[:, 2:3], 256), Ec_v[:, :])
        cat3 = jnp.concatenate([se, te, ce], axis=1)

        def fcb(i):
            return jnp.broadcast_to(bias[_B_FC[i]:_B_FC[i] + CC][None, :], (B, CC))

        def upb(i):
            w = _UP_OUT[i]
            return jnp.broadcast_to(bias[_B_UP[i]:_B_UP[i] + w][None, :], (B, w))

        c0 = se @ fc0_v[:, :] + fcb(0)
        c1 = _dot_t(jnp.concatenate([se, te], axis=1), fc1_v[:, :]) + fcb(1)
        c2 = _dot_t(cat3, fc2_v[:, :]) + fcb(2)
        c3 = _dot_t(cat3, fc3_v[:, :]) + fcb(3)
        c4 = _dot_t(cat3, fc4_v[:, :]) + fcb(4)

        x = z_v[:, :]
        x = _leaky(_dot_t(jnp.concatenate([x, c0], axis=1), up0_v[:, :]) + upb(0))
        x = _leaky(_dot_t(jnp.concatenate([x, c1], axis=1), up1_v[:, :]) + upb(1))
        x = _leaky(_dot_t(jnp.concatenate([x, c2], axis=1), up2_v[:, :]) + upb(2))
        x = _leaky(_dot_t(jnp.concatenate([x, c3], axis=1), up3_v[:, :]) + upb(3))
        y = jnp.concatenate([x, c4], axis=1) @ up4_v[:, :] + upb(4)
        y_ref[:, :] = y

    yv = y_ref[:, :]
    out_ref[:, :, :] = jnp.broadcast_to(yv[:, :, None], (B, CHUNK // 128, 128))


def kernel(z, studies, tasks, contrasts, Es, Et, Ec,
           fc0_W, fc0_b, fc1_W, fc1_b, fc2_W, fc2_b, fc3_W, fc3_b,
           fc4_W, fc4_b, up0_W, up0_b, up1_W, up1_b, up2_W, up2_b,
           up3_W, up3_b, up4_W, up4_b, idx0, idx1, idx2, idx3, idx4):
    idx2d = jnp.stack([studies, tasks, contrasts], axis=1)
    bias_all = jnp.concatenate([fc0_b, fc1_b, fc2_b, fc3_b, fc4_b,
                                up0_b, up1_b, up2_b, up3_b, up4_b])
    args = (idx2d, bias_all, z, Es.T, Et.T, Ec.T,
            fc0_W, fc1_W.T, fc2_W.T, fc3_W.T, fc4_W.T,
            up0_W.T, up1_W.T, up2_W.T, up3_W.T, up4_W)
    args = tuple(pltpu.with_memory_space_constraint(a, pltpu.MemorySpace.HBM)
                 for a in args)
    hbm = pl.BlockSpec(memory_space=pltpu.MemorySpace.HBM)
    out = pl.pallas_call(
        _fgl_kernel,
        grid=(N_CHUNKS,),
        in_specs=[hbm] * len(args),
        out_specs=pl.BlockSpec((B, CHUNK // 128, 128), lambda i: (0, i, 0)),
        out_shape=jax.ShapeDtypeStruct((B, N_OUT // 128, 128), jnp.float32),
        scratch_shapes=[
            pltpu.VMEM((B, 3), jnp.int32),
            pltpu.VMEM((201,), jnp.float32),
            pltpu.VMEM((B, ZS), jnp.float32),
            pltpu.VMEM((CC, 64), jnp.float32),
            pltpu.VMEM((CC, 128), jnp.float32),
            pltpu.VMEM((CC, 256), jnp.float32),
            pltpu.VMEM((CC, CC), jnp.float32),
            pltpu.VMEM((CC, 2 * CC), jnp.float32),
            pltpu.VMEM((CC, 3 * CC), jnp.float32),
            pltpu.VMEM((CC, 3 * CC), jnp.float32),
            pltpu.VMEM((CC, 3 * CC), jnp.float32),
            pltpu.VMEM((64, ZS + CC), jnp.float32),
            pltpu.VMEM((32, 64 + CC), jnp.float32),
            pltpu.VMEM((16, 32 + CC), jnp.float32),
            pltpu.VMEM((8, 16 + CC), jnp.float32),
            pltpu.VMEM((8 + CC, 1), jnp.float32),
            pltpu.VMEM((B, 1), jnp.float32),
            pltpu.SemaphoreType.DMA((16,)),
        ],
    )(*args)
    return out.reshape(B, N_OUT, 1)


# speedup vs baseline: 3.3201x; 1.0762x over previous
import jax
import jax.numpy as jnp
from jax.experimental import pallas as pl
from jax.experimental.pallas import tpu as pltpu

B = 32
ZS = 128
CC = 16
N_OUT = 65536
N_CHUNKS = 8
CHUNK = N_OUT // N_CHUNKS

_B_FC = [0, 16, 32, 48, 64]
_B_UP = [80, 144, 176, 192, 200]
_UP_OUT = [64, 32, 16, 8, 1]
_P_UP4W = 201
_P_IDX = [225, 257, 289]
_P_LEN = 321


def _leaky(x):
    return jnp.where(x > 0, x, 0.2 * x)


def _dot_t(a, b_t):
    return jax.lax.dot_general(a, b_t, (((1,), (1,)), ((), ())),
                               preferred_element_type=jnp.float32)


def _fgl_kernel(pk_hbm, z_hbm, Es_hbm, Et_hbm, Ec_hbm,
                fc0_hbm, fc1_hbm, fc2_hbm, fc3_hbm, fc4_hbm,
                up0_hbm, up1_hbm, up2_hbm, up3_hbm,
                out_ref,
                pk_v, z_v, Es_v, Et_v, Ec_v,
                fc0_v, fc1_v, fc2_v, fc3_v, fc4_v,
                up0_v, up1_v, up2_v, up3_v,
                y_ref, sem):
    f32 = jnp.float32

    @pl.when(pl.program_id(0) == 0)
    def _compute_chain():
        pairs = [(pk_hbm, pk_v), (z_hbm, z_v),
                 (Es_hbm, Es_v), (Et_hbm, Et_v), (Ec_hbm, Ec_v),
                 (fc0_hbm, fc0_v), (fc1_hbm, fc1_v), (fc2_hbm, fc2_v),
                 (fc3_hbm, fc3_v), (fc4_hbm, fc4_v),
                 (up0_hbm, up0_v), (up1_hbm, up1_v), (up2_hbm, up2_v),
                 (up3_hbm, up3_v)]
        copies = [pltpu.make_async_copy(s, d, sem.at[i])
                  for i, (s, d) in enumerate(pairs)]
        for c in copies:
            c.start()
        for c in copies:
            c.wait()

        bias = pk_v[:]

        def idx_col(i):
            return jnp.transpose(bias[_P_IDX[i]:_P_IDX[i] + B].reshape(1, B))

        def onehot(col, n):
            iota = jax.lax.broadcasted_iota(jnp.int32, (B, n), 1)
            return (iota == col.astype(jnp.int32)).astype(f32)

        se = _dot_t(onehot(idx_col(0), 64), Es_v[:, :])
        te = _dot_t(onehot(idx_col(1), 128), Et_v[:, :])
        ce = _dot_t(onehot(idx_col(2), 256), Ec_v[:, :])
        cat3 = jnp.concatenate([se, te, ce], axis=1)

        def fcb(i):
            return jnp.broadcast_to(bias[_B_FC[i]:_B_FC[i] + CC][None, :], (B, CC))

        def upb(i):
            w = _UP_OUT[i]
            return jnp.broadcast_to(bias[_B_UP[i]:_B_UP[i] + w][None, :], (B, w))

        c0 = se @ fc0_v[:, :] + fcb(0)
        c1 = _dot_t(jnp.concatenate([se, te], axis=1), fc1_v[:, :]) + fcb(1)
        c2 = _dot_t(cat3, fc2_v[:, :]) + fcb(2)
        c3 = _dot_t(cat3, fc3_v[:, :]) + fcb(3)
        c4 = _dot_t(cat3, fc4_v[:, :]) + fcb(4)

        x = z_v[:, :]
        x = _leaky(_dot_t(jnp.concatenate([x, c0], axis=1), up0_v[:, :]) + upb(0))
        x = _leaky(_dot_t(jnp.concatenate([x, c1], axis=1), up1_v[:, :]) + upb(1))
        x = _leaky(_dot_t(jnp.concatenate([x, c2], axis=1), up2_v[:, :]) + upb(2))
        x = _leaky(_dot_t(jnp.concatenate([x, c3], axis=1), up3_v[:, :]) + upb(3))
        w4 = jnp.broadcast_to(bias[_P_UP4W:_P_UP4W + 24][None, :], (B, 24))
        x4 = jnp.concatenate([x, c4], axis=1)
        y = jnp.sum(x4 * w4, axis=1, keepdims=True) + upb(4)
        y_ref[:, :] = y

    yv = y_ref[:, :]
    out_ref[:, :, :] = jnp.broadcast_to(yv[:, :, None], (B, CHUNK // 128, 128))


def kernel(z, studies, tasks, contrasts, Es, Et, Ec,
           fc0_W, fc0_b, fc1_W, fc1_b, fc2_W, fc2_b, fc3_W, fc3_b,
           fc4_W, fc4_b, up0_W, up0_b, up1_W, up1_b, up2_W, up2_b,
           up3_W, up3_b, up4_W, up4_b, idx0, idx1, idx2, idx3, idx4):
    f32 = jnp.float32
    packed = jnp.concatenate(
        [fc0_b, fc1_b, fc2_b, fc3_b, fc4_b,
         up0_b, up1_b, up2_b, up3_b, up4_b,
         up4_W.ravel(),
         studies.astype(f32), tasks.astype(f32), contrasts.astype(f32)])
    args = (packed, z, Es.T, Et.T, Ec.T,
            fc0_W, fc1_W.T, fc2_W.T, fc3_W.T, fc4_W.T,
            up0_W.T, up1_W.T, up2_W.T, up3_W.T)
    args = tuple(pltpu.with_memory_space_constraint(a, pltpu.MemorySpace.HBM)
                 for a in args)
    hbm = pl.BlockSpec(memory_space=pltpu.MemorySpace.HBM)
    out = pl.pallas_call(
        _fgl_kernel,
        grid=(N_CHUNKS,),
        in_specs=[hbm] * len(args),
        out_specs=pl.BlockSpec((B, CHUNK // 128, 128), lambda i: (0, i, 0)),
        out_shape=jax.ShapeDtypeStruct((B, N_OUT // 128, 128), jnp.float32),
        scratch_shapes=[
            pltpu.VMEM((_P_LEN,), jnp.float32),
            pltpu.VMEM((B, ZS), jnp.float32),
            pltpu.VMEM((CC, 64), jnp.float32),
            pltpu.VMEM((CC, 128), jnp.float32),
            pltpu.VMEM((CC, 256), jnp.float32),
            pltpu.VMEM((CC, CC), jnp.float32),
            pltpu.VMEM((CC, 2 * CC), jnp.float32),
            pltpu.VMEM((CC, 3 * CC), jnp.float32),
            pltpu.VMEM((CC, 3 * CC), jnp.float32),
            pltpu.VMEM((CC, 3 * CC), jnp.float32),
            pltpu.VMEM((64, ZS + CC), jnp.float32),
            pltpu.VMEM((32, 64 + CC), jnp.float32),
            pltpu.VMEM((16, 32 + CC), jnp.float32),
            pltpu.VMEM((8, 16 + CC), jnp.float32),
            pltpu.VMEM((B, 1), jnp.float32),
            pltpu.SemaphoreType.DMA((14,)),
        ],
    )(*args)
    return out.reshape(B, N_OUT, 1)


# speedup vs baseline: 4.6440x; 1.3988x over previous
import jax
import jax.numpy as jnp
from jax.experimental import pallas as pl
from jax.experimental.pallas import tpu as pltpu

B = 32
ZS = 128
CC = 16
N_OUT = 65536
N_CHUNKS = 8
CHUNK = N_OUT // N_CHUNKS


def _leaky(x):
    return jnp.where(x > 0, x, 0.2 * x)


def _dot_t(a, b_t):
    return jax.lax.dot_general(a, b_t, (((1,), (1,)), ((), ())),
                               preferred_element_type=jnp.float32)


def _fgl_kernel(s_hbm, t_hbm, c_hbm,
                fc0b_hbm, fc1b_hbm, fc2b_hbm, fc3b_hbm, fc4b_hbm,
                up0b_hbm, up1b_hbm, up2b_hbm, up3b_hbm, up4b_hbm,
                up4w_hbm, z_hbm, Es_hbm, Et_hbm, Ec_hbm,
                fc0_hbm, fc1_hbm, fc2_hbm, fc3_hbm, fc4_hbm,
                up0_hbm, up1_hbm, up2_hbm, up3_hbm,
                out_ref,
                s_v, t_v, c_v,
                fc0b_v, fc1b_v, fc2b_v, fc3b_v, fc4b_v,
                up0b_v, up1b_v, up2b_v, up3b_v, up4b_v,
                up4w_v, z_v, Es_v, Et_v, Ec_v,
                fc0_v, fc1_v, fc2_v, fc3_v, fc4_v,
                up0_v, up1_v, up2_v, up3_v,
                y_ref, sem):
    f32 = jnp.float32

    @pl.when(pl.program_id(0) == 0)
    def _compute_chain():
        hbm_refs = [s_hbm, t_hbm, c_hbm,
                    fc0b_hbm, fc1b_hbm, fc2b_hbm, fc3b_hbm, fc4b_hbm,
                    up0b_hbm, up1b_hbm, up2b_hbm, up3b_hbm, up4b_hbm,
                    up4w_hbm, z_hbm, Es_hbm, Et_hbm, Ec_hbm,
                    fc0_hbm, fc1_hbm, fc2_hbm, fc3_hbm, fc4_hbm,
                    up0_hbm, up1_hbm, up2_hbm, up3_hbm]
        vmem_refs = [s_v, t_v, c_v,
                     fc0b_v, fc1b_v, fc2b_v, fc3b_v, fc4b_v,
                     up0b_v, up1b_v, up2b_v, up3b_v, up4b_v,
                     up4w_v, z_v, Es_v, Et_v, Ec_v,
                     fc0_v, fc1_v, fc2_v, fc3_v, fc4_v,
                     up0_v, up1_v, up2_v, up3_v]
        copies = [pltpu.make_async_copy(s, d, sem.at[i])
                  for i, (s, d) in enumerate(zip(hbm_refs, vmem_refs))]
        for c in copies:
            c.start()
        for c in copies:
            c.wait()

        def idx_col(ref):
            return jnp.transpose(ref[:].reshape(1, B))

        def onehot(col, n):
            iota = jax.lax.broadcasted_iota(jnp.int32, (B, n), 1)
            return (iota == col).astype(f32)

        def rowb(ref, w):
            return jnp.broadcast_to(ref[:][None, :], (B, w))

        se = _dot_t(onehot(idx_col(s_v), 64), Es_v[:, :])
        te = _dot_t(onehot(idx_col(t_v), 128), Et_v[:, :])
        ce = _dot_t(onehot(idx_col(c_v), 256), Ec_v[:, :])
        cat3 = jnp.concatenate([se, te, ce], axis=1)

        c0 = se @ fc0_v[:, :] + rowb(fc0b_v, CC)
        c1 = _dot_t(jnp.concatenate([se, te], axis=1), fc1_v[:, :]) + rowb(fc1b_v, CC)
        c2 = _dot_t(cat3, fc2_v[:, :]) + rowb(fc2b_v, CC)
        c3 = _dot_t(cat3, fc3_v[:, :]) + rowb(fc3b_v, CC)
        c4 = _dot_t(cat3, fc4_v[:, :]) + rowb(fc4b_v, CC)

        x = z_v[:, :]
        x = _leaky(_dot_t(jnp.concatenate([x, c0], axis=1), up0_v[:, :]) + rowb(up0b_v, 64))
        x = _leaky(_dot_t(jnp.concatenate([x, c1], axis=1), up1_v[:, :]) + rowb(up1b_v, 32))
        x = _leaky(_dot_t(jnp.concatenate([x, c2], axis=1), up2_v[:, :]) + rowb(up2b_v, 16))
        x = _leaky(_dot_t(jnp.concatenate([x, c3], axis=1), up3_v[:, :]) + rowb(up3b_v, 8))
        x4 = jnp.concatenate([x, c4], axis=1)
        w4 = jnp.broadcast_to(up4w_v[:][None, :], (B, 24))
        y = jnp.sum(x4 * w4, axis=1, keepdims=True) + rowb(up4b_v, 1)
        y_ref[:, :] = y

    yv = y_ref[:, :]
    out_ref[:, :, :] = jnp.broadcast_to(yv[:, :, None], (B, CHUNK // 128, 128))


def kernel(z, studies, tasks, contrasts, Es, Et, Ec,
           fc0_W, fc0_b, fc1_W, fc1_b, fc2_W, fc2_b, fc3_W, fc3_b,
           fc4_W, fc4_b, up0_W, up0_b, up1_W, up1_b, up2_W, up2_b,
           up3_W, up3_b, up4_W, up4_b, idx0, idx1, idx2, idx3, idx4):
    args = (studies, tasks, contrasts,
            fc0_b, fc1_b, fc2_b, fc3_b, fc4_b,
            up0_b, up1_b, up2_b, up3_b, up4_b,
            up4_W.ravel(), z, Es.T, Et.T, Ec.T,
            fc0_W, fc1_W.T, fc2_W.T, fc3_W.T, fc4_W.T,
            up0_W.T, up1_W.T, up2_W.T, up3_W.T)
    args = tuple(a if a.size == 1
                 else pltpu.with_memory_space_constraint(a, pltpu.MemorySpace.HBM)
                 for a in args)
    hbm = pl.BlockSpec(memory_space=pltpu.MemorySpace.HBM)
    out = pl.pallas_call(
        _fgl_kernel,
        grid=(N_CHUNKS,),
        in_specs=[hbm] * len(args),
        out_specs=pl.BlockSpec((B, CHUNK // 128, 128), lambda i: (0, i, 0)),
        out_shape=jax.ShapeDtypeStruct((B, N_OUT // 128, 128), jnp.float32),
        scratch_shapes=[
            pltpu.VMEM((B,), jnp.int32),
            pltpu.VMEM((B,), jnp.int32),
            pltpu.VMEM((B,), jnp.int32),
            pltpu.VMEM((CC,), jnp.float32),
            pltpu.VMEM((CC,), jnp.float32),
            pltpu.VMEM((CC,), jnp.float32),
            pltpu.VMEM((CC,), jnp.float32),
            pltpu.VMEM((CC,), jnp.float32),
            pltpu.VMEM((64,), jnp.float32),
            pltpu.VMEM((32,), jnp.float32),
            pltpu.VMEM((16,), jnp.float32),
            pltpu.VMEM((8,), jnp.float32),
            pltpu.VMEM((1,), jnp.float32),
            pltpu.VMEM((24,), jnp.float32),
            pltpu.VMEM((B, ZS), jnp.float32),
            pltpu.VMEM((CC, 64), jnp.float32),
            pltpu.VMEM((CC, 128), jnp.float32),
            pltpu.VMEM((CC, 256), jnp.float32),
            pltpu.VMEM((CC, CC), jnp.float32),
            pltpu.VMEM((CC, 2 * CC), jnp.float32),
            pltpu.VMEM((CC, 3 * CC), jnp.float32),
            pltpu.VMEM((CC, 3 * CC), jnp.float32),
            pltpu.VMEM((CC, 3 * CC), jnp.float32),
            pltpu.VMEM((64, ZS + CC), jnp.float32),
            pltpu.VMEM((32, 64 + CC), jnp.float32),
            pltpu.VMEM((16, 32 + CC), jnp.float32),
            pltpu.VMEM((8, 16 + CC), jnp.float32),
            pltpu.VMEM((B, 1), jnp.float32),
            pltpu.SemaphoreType.DMA((27,)),
        ],
    )(*args)
    return out.reshape(B, N_OUT, 1)


# speedup vs baseline: 5.5129x; 1.1871x over previous
import jax
import jax.numpy as jnp
from jax.experimental import pallas as pl
from jax.experimental.pallas import tpu as pltpu

B = 32
ZS = 128
CC = 16
N_OUT = 65536
N_CHUNKS = 4
CHUNK = N_OUT // N_CHUNKS


def _leaky(x):
    return jnp.where(x > 0, x, 0.2 * x)


def _dot_t(a, b_t):
    return jax.lax.dot_general(a, b_t, (((1,), (1,)), ((), ())),
                               preferred_element_type=jnp.float32)


def _fgl_kernel(s_hbm, t_hbm, c_hbm,
                fc0b_hbm, fc1b_hbm, fc2b_hbm, fc3b_hbm, fc4b_hbm,
                up0b_hbm, up1b_hbm, up2b_hbm, up3b_hbm, up4b_hbm,
                up4w_hbm, z_hbm, Es_hbm, Et_hbm, Ec_hbm,
                fc0_hbm, fc1_hbm, fc2_hbm, fc3_hbm, fc4_hbm,
                up0_hbm, up1_hbm, up2_hbm, up3_hbm,
                out_ref,
                s_v, t_v, c_v,
                fc0b_v, fc1b_v, fc2b_v, fc3b_v, fc4b_v,
                up0b_v, up1b_v, up2b_v, up3b_v, up4b_v,
                up4w_v, z_v, Es_v, Et_v, Ec_v,
                fc0_v, fc1_v, fc2_v, fc3_v, fc4_v,
                up0_v, up1_v, up2_v, up3_v,
                y_ref, sem):
    f32 = jnp.float32

    @pl.when(pl.program_id(0) == 0)
    def _compute_chain():
        hbm_refs = [s_hbm, t_hbm, c_hbm,
                    fc0b_hbm, fc1b_hbm, fc2b_hbm, fc3b_hbm, fc4b_hbm,
                    up0b_hbm, up1b_hbm, up2b_hbm, up3b_hbm, up4b_hbm,
                    up4w_hbm, z_hbm, Es_hbm, Et_hbm, Ec_hbm,
                    fc0_hbm, fc1_hbm, fc2_hbm, fc3_hbm, fc4_hbm,
                    up0_hbm, up1_hbm, up2_hbm, up3_hbm]
        vmem_refs = [s_v, t_v, c_v,
                     fc0b_v, fc1b_v, fc2b_v, fc3b_v, fc4b_v,
                     up0b_v, up1b_v, up2b_v, up3b_v, up4b_v,
                     up4w_v, z_v, Es_v, Et_v, Ec_v,
                     fc0_v, fc1_v, fc2_v, fc3_v, fc4_v,
                     up0_v, up1_v, up2_v, up3_v]
        copies = [pltpu.make_async_copy(s, d, sem.at[i])
                  for i, (s, d) in enumerate(zip(hbm_refs, vmem_refs))]
        for c in copies:
            c.start()
        for c in copies:
            c.wait()

        def idx_col(ref):
            return jnp.transpose(ref[:].reshape(1, B))

        def onehot(col, n):
            iota = jax.lax.broadcasted_iota(jnp.int32, (B, n), 1)
            return (iota == col).astype(f32)

        def rowb(ref, w):
            return jnp.broadcast_to(ref[:][None, :], (B, w))

        se = _dot_t(onehot(idx_col(s_v), 64), Es_v[:, :])
        te = _dot_t(onehot(idx_col(t_v), 128), Et_v[:, :])
        ce = _dot_t(onehot(idx_col(c_v), 256), Ec_v[:, :])
        cat3 = jnp.concatenate([se, te, ce], axis=1)

        c0 = se @ fc0_v[:, :] + rowb(fc0b_v, CC)
        c1 = _dot_t(jnp.concatenate([se, te], axis=1), fc1_v[:, :]) + rowb(fc1b_v, CC)
        c2 = _dot_t(cat3, fc2_v[:, :]) + rowb(fc2b_v, CC)
        c3 = _dot_t(cat3, fc3_v[:, :]) + rowb(fc3b_v, CC)
        c4 = _dot_t(cat3, fc4_v[:, :]) + rowb(fc4b_v, CC)

        x = z_v[:, :]
        x = _leaky(_dot_t(jnp.concatenate([x, c0], axis=1), up0_v[:, :]) + rowb(up0b_v, 64))
        x = _leaky(_dot_t(jnp.concatenate([x, c1], axis=1), up1_v[:, :]) + rowb(up1b_v, 32))
        x = _leaky(_dot_t(jnp.concatenate([x, c2], axis=1), up2_v[:, :]) + rowb(up2b_v, 16))
        x = _leaky(_dot_t(jnp.concatenate([x, c3], axis=1), up3_v[:, :]) + rowb(up3b_v, 8))
        x4 = jnp.concatenate([x, c4], axis=1)
        w4 = jnp.broadcast_to(up4w_v[:][None, :], (B, 24))
        y = jnp.sum(x4 * w4, axis=1, keepdims=True) + rowb(up4b_v, 1)
        y_ref[:, :] = y

    yv = y_ref[:, :]
    out_ref[:, :, :] = jnp.broadcast_to(yv[:, :, None], (B, CHUNK // 128, 128))


def kernel(z, studies, tasks, contrasts, Es, Et, Ec,
           fc0_W, fc0_b, fc1_W, fc1_b, fc2_W, fc2_b, fc3_W, fc3_b,
           fc4_W, fc4_b, up0_W, up0_b, up1_W, up1_b, up2_W, up2_b,
           up3_W, up3_b, up4_W, up4_b, idx0, idx1, idx2, idx3, idx4):
    args = (studies, tasks, contrasts,
            fc0_b, fc1_b, fc2_b, fc3_b, fc4_b,
            up0_b, up1_b, up2_b, up3_b, up4_b,
            up4_W.ravel(), z, Es.T, Et.T, Ec.T,
            fc0_W, fc1_W.T, fc2_W.T, fc3_W.T, fc4_W.T,
            up0_W.T, up1_W.T, up2_W.T, up3_W.T)
    args = tuple(a if a.size == 1
                 else pltpu.with_memory_space_constraint(a, pltpu.MemorySpace.HBM)
                 for a in args)
    hbm = pl.BlockSpec(memory_space=pltpu.MemorySpace.HBM)
    out = pl.pallas_call(
        _fgl_kernel,
        grid=(N_CHUNKS,),
        in_specs=[hbm] * len(args),
        out_specs=pl.BlockSpec((B, CHUNK // 128, 128), lambda i: (0, i, 0)),
        out_shape=jax.ShapeDtypeStruct((B, N_OUT // 128, 128), jnp.float32),
        scratch_shapes=[
            pltpu.VMEM((B,), jnp.int32),
            pltpu.VMEM((B,), jnp.int32),
            pltpu.VMEM((B,), jnp.int32),
            pltpu.VMEM((CC,), jnp.float32),
            pltpu.VMEM((CC,), jnp.float32),
            pltpu.VMEM((CC,), jnp.float32),
            pltpu.VMEM((CC,), jnp.float32),
            pltpu.VMEM((CC,), jnp.float32),
            pltpu.VMEM((64,), jnp.float32),
            pltpu.VMEM((32,), jnp.float32),
            pltpu.VMEM((16,), jnp.float32),
            pltpu.VMEM((8,), jnp.float32),
            pltpu.VMEM((1,), jnp.float32),
            pltpu.VMEM((24,), jnp.float32),
            pltpu.VMEM((B, ZS), jnp.float32),
            pltpu.VMEM((CC, 64), jnp.float32),
            pltpu.VMEM((CC, 128), jnp.float32),
            pltpu.VMEM((CC, 256), jnp.float32),
            pltpu.VMEM((CC, CC), jnp.float32),
            pltpu.VMEM((CC, 2 * CC), jnp.float32),
            pltpu.VMEM((CC, 3 * CC), jnp.float32),
            pltpu.VMEM((CC, 3 * CC), jnp.float32),
            pltpu.VMEM((CC, 3 * CC), jnp.float32),
            pltpu.VMEM((64, ZS + CC), jnp.float32),
            pltpu.VMEM((32, 64 + CC), jnp.float32),
            pltpu.VMEM((16, 32 + CC), jnp.float32),
            pltpu.VMEM((8, 16 + CC), jnp.float32),
            pltpu.VMEM((B, 1), jnp.float32),
            pltpu.SemaphoreType.DMA((27,)),
        ],
    )(*args)
    return out.reshape(B, N_OUT, 1)


# speedup vs baseline: 5.6604x; 1.0268x over previous
import jax
import jax.numpy as jnp
from jax.experimental import pallas as pl
from jax.experimental.pallas import tpu as pltpu

B = 32
ZS = 128
CC = 16
N_OUT = 65536
N_CHUNKS = 2
CHUNK = N_OUT // N_CHUNKS


def _leaky(x):
    return jnp.where(x > 0, x, 0.2 * x)


def _dot_t(a, b_t):
    return jax.lax.dot_general(a, b_t, (((1,), (1,)), ((), ())),
                               preferred_element_type=jnp.float32)


def _fgl_kernel(s_hbm, t_hbm, c_hbm,
                fc0b_hbm, fc1b_hbm, fc2b_hbm, fc3b_hbm, fc4b_hbm,
                up0b_hbm, up1b_hbm, up2b_hbm, up3b_hbm, up4b_hbm,
                up4w_hbm, z_hbm, Es_hbm, Et_hbm, Ec_hbm,
                fc0_hbm, fc1_hbm, fc2_hbm, fc3_hbm, fc4_hbm,
                up0_hbm, up1_hbm, up2_hbm, up3_hbm,
                out_ref,
                s_v, t_v, c_v,
                fc0b_v, fc1b_v, fc2b_v, fc3b_v, fc4b_v,
                up0b_v, up1b_v, up2b_v, up3b_v, up4b_v,
                up4w_v, z_v, Es_v, Et_v, Ec_v,
                fc0_v, fc1_v, fc2_v, fc3_v, fc4_v,
                up0_v, up1_v, up2_v, up3_v,
                y_ref, sem):
    f32 = jnp.float32

    @pl.when(pl.program_id(0) == 0)
    def _compute_chain():
        hbm_refs = [s_hbm, t_hbm, c_hbm,
                    fc0b_hbm, fc1b_hbm, fc2b_hbm, fc3b_hbm, fc4b_hbm,
                    up0b_hbm, up1b_hbm, up2b_hbm, up3b_hbm, up4b_hbm,
                    up4w_hbm, z_hbm, Es_hbm, Et_hbm, Ec_hbm,
                    fc0_hbm, fc1_hbm, fc2_hbm, fc3_hbm, fc4_hbm,
                    up0_hbm, up1_hbm, up2_hbm, up3_hbm]
        vmem_refs = [s_v, t_v, c_v,
                     fc0b_v, fc1b_v, fc2b_v, fc3b_v, fc4b_v,
                     up0b_v, up1b_v, up2b_v, up3b_v, up4b_v,
                     up4w_v, z_v, Es_v, Et_v, Ec_v,
                     fc0_v, fc1_v, fc2_v, fc3_v, fc4_v,
                     up0_v, up1_v, up2_v, up3_v]
        copies = [pltpu.make_async_copy(s, d, sem.at[i])
                  for i, (s, d) in enumerate(zip(hbm_refs, vmem_refs))]
        for c in copies:
            c.start()
        for c in copies:
            c.wait()

        def idx_col(ref):
            return jnp.transpose(ref[:].reshape(1, B))

        def onehot(col, n):
            iota = jax.lax.broadcasted_iota(jnp.int32, (B, n), 1)
            return (iota == col).astype(f32)

        def rowb(ref, w):
            return jnp.broadcast_to(ref[:][None, :], (B, w))

        se = _dot_t(onehot(idx_col(s_v), 64), Es_v[:, :])
        te = _dot_t(onehot(idx_col(t_v), 128), Et_v[:, :])
        ce = _dot_t(onehot(idx_col(c_v), 256), Ec_v[:, :])
        cat3 = jnp.concatenate([se, te, ce], axis=1)

        c0 = se @ fc0_v[:, :] + rowb(fc0b_v, CC)
        c1 = _dot_t(jnp.concatenate([se, te], axis=1), fc1_v[:, :]) + rowb(fc1b_v, CC)
        c2 = _dot_t(cat3, fc2_v[:, :]) + rowb(fc2b_v, CC)
        c3 = _dot_t(cat3, fc3_v[:, :]) + rowb(fc3b_v, CC)
        c4 = _dot_t(cat3, fc4_v[:, :]) + rowb(fc4b_v, CC)

        x = z_v[:, :]
        x = _leaky(_dot_t(jnp.concatenate([x, c0], axis=1), up0_v[:, :]) + rowb(up0b_v, 64))
        x = _leaky(_dot_t(jnp.concatenate([x, c1], axis=1), up1_v[:, :]) + rowb(up1b_v, 32))
        x = _leaky(_dot_t(jnp.concatenate([x, c2], axis=1), up2_v[:, :]) + rowb(up2b_v, 16))
        x = _leaky(_dot_t(jnp.concatenate([x, c3], axis=1), up3_v[:, :]) + rowb(up3b_v, 8))
        x4 = jnp.concatenate([x, c4], axis=1)
        w4 = jnp.broadcast_to(up4w_v[:][None, :], (B, 24))
        y = jnp.sum(x4 * w4, axis=1, keepdims=True) + rowb(up4b_v, 1)
        y_ref[:, :] = y

    yv = y_ref[:, :]
    out_ref[:, :, :] = jnp.broadcast_to(yv[:, :, None], (B, CHUNK // 128, 128))


def kernel(z, studies, tasks, contrasts, Es, Et, Ec,
           fc0_W, fc0_b, fc1_W, fc1_b, fc2_W, fc2_b, fc3_W, fc3_b,
           fc4_W, fc4_b, up0_W, up0_b, up1_W, up1_b, up2_W, up2_b,
           up3_W, up3_b, up4_W, up4_b, idx0, idx1, idx2, idx3, idx4):
    args = (studies, tasks, contrasts,
            fc0_b, fc1_b, fc2_b, fc3_b, fc4_b,
            up0_b, up1_b, up2_b, up3_b, up4_b,
            up4_W.ravel(), z, Es.T, Et.T, Ec.T,
            fc0_W, fc1_W.T, fc2_W.T, fc3_W.T, fc4_W.T,
            up0_W.T, up1_W.T, up2_W.T, up3_W.T)
    args = tuple(a if a.size == 1
                 else pltpu.with_memory_space_constraint(a, pltpu.MemorySpace.HBM)
                 for a in args)
    hbm = pl.BlockSpec(memory_space=pltpu.MemorySpace.HBM)
    out = pl.pallas_call(
        _fgl_kernel,
        grid=(N_CHUNKS,),
        in_specs=[hbm] * len(args),
        out_specs=pl.BlockSpec((B, CHUNK // 128, 128), lambda i: (0, i, 0)),
        out_shape=jax.ShapeDtypeStruct((B, N_OUT // 128, 128), jnp.float32),
        scratch_shapes=[
            pltpu.VMEM((B,), jnp.int32),
            pltpu.VMEM((B,), jnp.int32),
            pltpu.VMEM((B,), jnp.int32),
            pltpu.VMEM((CC,), jnp.float32),
            pltpu.VMEM((CC,), jnp.float32),
            pltpu.VMEM((CC,), jnp.float32),
            pltpu.VMEM((CC,), jnp.float32),
            pltpu.VMEM((CC,), jnp.float32),
            pltpu.VMEM((64,), jnp.float32),
            pltpu.VMEM((32,), jnp.float32),
            pltpu.VMEM((16,), jnp.float32),
            pltpu.VMEM((8,), jnp.float32),
            pltpu.VMEM((1,), jnp.float32),
            pltpu.VMEM((24,), jnp.float32),
            pltpu.VMEM((B, ZS), jnp.float32),
            pltpu.VMEM((CC, 64), jnp.float32),
            pltpu.VMEM((CC, 128), jnp.float32),
            pltpu.VMEM((CC, 256), jnp.float32),
            pltpu.VMEM((CC, CC), jnp.float32),
            pltpu.VMEM((CC, 2 * CC), jnp.float32),
            pltpu.VMEM((CC, 3 * CC), jnp.float32),
            pltpu.VMEM((CC, 3 * CC), jnp.float32),
            pltpu.VMEM((CC, 3 * CC), jnp.float32),
            pltpu.VMEM((64, ZS + CC), jnp.float32),
            pltpu.VMEM((32, 64 + CC), jnp.float32),
            pltpu.VMEM((16, 32 + CC), jnp.float32),
            pltpu.VMEM((8, 16 + CC), jnp.float32),
            pltpu.VMEM((B, 1), jnp.float32),
            pltpu.SemaphoreType.DMA((27,)),
        ],
    )(*args)
    return out.reshape(B, N_OUT, 1)
